# baseline (speedup 1.0000x reference)
import functools
import math

import jax
import jax.numpy as jnp
from jax import lax
from jax.experimental import pallas as pl
from jax.experimental.pallas import tpu as pltpu

H = 1024
RATIO = 4
SELK = 512
WIN = 256
SCALE = 1.0 / math.sqrt(H // 16)
TILE = 256


def _cp(ndims):
    return pltpu.CompilerParams(dimension_semantics=("parallel",) * ndims)


def _softmax(s):
    m = jnp.max(s, axis=-1, keepdims=True)
    e = jnp.exp(s - m)
    return e / jnp.sum(e, axis=-1, keepdims=True)


def _qkv_score_body(x_ref, wq, bq, wk, bk, wv, bv, ws, bs,
                    q_out, k_out, v_out, s_out):
    x = x_ref[0]
    x16 = x.astype(jnp.bfloat16)
    q_out[0] = jnp.dot(x16, wq[...], preferred_element_type=jnp.float32) + bq[0]
    k_out[0] = jnp.dot(x16, wk[...], preferred_element_type=jnp.float32) + bk[0]
    v_out[0] = jnp.dot(x16, wv[...], preferred_element_type=jnp.float32) + bv[0]
    s_out[0] = lax.dot_general(ws[...], x, (((1,), (1,)), ((), ())),
                               preferred_element_type=jnp.float32) + bs[...]


def _qkv_body(x_ref, wq, bq, wk, bk, wv, bv, q_out, k_out, v_out):
    x16 = x_ref[0].astype(jnp.bfloat16)
    q_out[0] = jnp.dot(x16, wq[...], preferred_element_type=jnp.float32) + bq[0]
    k_out[0] = jnp.dot(x16, wk[...], preferred_element_type=jnp.float32) + bk[0]
    v_out[0] = jnp.dot(x16, wv[...], preferred_element_type=jnp.float32) + bv[0]


def _w_spec(shape):
    return pl.BlockSpec(shape, lambda b, t: (0,) * len(shape))


def _row_spec(n):
    return pl.BlockSpec((1, n, H), lambda b, t: (b, t, 0))


def _qkv_score(x, Wq, bq, Wk, bk, Wv, bv, Wst, bs):
    B, S, _ = x.shape
    grid = (B, S // TILE)
    out = [jax.ShapeDtypeStruct((B, S, H), jnp.float32)] * 3 + [
        jax.ShapeDtypeStruct((B, 1, S), jnp.float32)]
    return pl.pallas_call(
        _qkv_score_body,
        grid=grid,
        compiler_params=_cp(2),
        in_specs=[
            _row_spec(TILE),
            _w_spec((H, H)), _w_spec((1, H)),
            _w_spec((H, H)), _w_spec((1, H)),
            _w_spec((H, H)), _w_spec((1, H)),
            _w_spec((1, H)), _w_spec((1, 1)),
        ],
        out_specs=[_row_spec(TILE), _row_spec(TILE), _row_spec(TILE),
                   pl.BlockSpec((1, 1, TILE), lambda b, t: (b, 0, t))],
        out_shape=out,
    )(x, Wq, bq, Wk, bk, Wv, bv, Wst, bs)


def _qkv(x, Wq, bq, Wk, bk, Wv, bv):
    B, S, _ = x.shape
    grid = (B, S // TILE)
    out = [jax.ShapeDtypeStruct((B, S, H), jnp.float32)] * 3
    return pl.pallas_call(
        _qkv_body,
        grid=grid,
        compiler_params=_cp(2),
        in_specs=[
            _row_spec(TILE),
            _w_spec((H, H)), _w_spec((1, H)),
            _w_spec((H, H)), _w_spec((1, H)),
            _w_spec((H, H)), _w_spec((1, H)),
        ],
        out_specs=[_row_spec(TILE)] * 3,
        out_shape=out,
    )(x, Wq, bq, Wk, bk, Wv, bv)


def _cproj_body(x_ref, wc, bc, out_ref):
    out_ref[0] = jnp.dot(x_ref[0].astype(jnp.bfloat16), wc[...],
                         preferred_element_type=jnp.float32) + bc[0]


def _compress(blocks, Wc, bc):
    B, NB, D = blocks.shape
    grid = (B, NB // TILE)
    return pl.pallas_call(
        _cproj_body,
        grid=grid,
        compiler_params=_cp(2),
        in_specs=[pl.BlockSpec((1, TILE, D), lambda b, t: (b, t, 0)),
                  _w_spec((D, H)), _w_spec((1, H))],
        out_specs=_row_spec(TILE),
        out_shape=jax.ShapeDtypeStruct((B, NB, H), jnp.float32),
    )(blocks, Wc, bc)


def _attn_body(q_ref, k_ref, v_ref, o_ref):
    s = jnp.dot(q_ref[0], k_ref[0].T, preferred_element_type=jnp.float32) * SCALE
    o_ref[0] = jnp.dot(_softmax(s), v_ref[0], preferred_element_type=jnp.float32)


def _attn(q, k, v):
    B, N, _ = q.shape
    spec = pl.BlockSpec((1, N, H), lambda b: (b, 0, 0))
    return pl.pallas_call(
        _attn_body,
        grid=(B,),
        compiler_params=_cp(1),
        in_specs=[spec, spec, spec],
        out_specs=spec,
        out_shape=jax.ShapeDtypeStruct((B, N, H), jnp.float32),
    )(q, k, v)


def _excl_prefix(f):
    S = f.shape[1]
    x = f
    k = 1
    while k < S:
        x = x + jnp.concatenate([jnp.zeros((1, k), f.dtype), x[:, :-k]], axis=1)
        k *= 2
    return x - f


def _select_body(s_ref, p_ref):
    x = s_ref[0]
    kf = float(SELK)

    lo0 = jnp.min(x)
    hi0 = jnp.max(x) + 1.0

    def body(_, lohi):
        lo, hi = lohi
        mid = (lo + hi) * 0.5
        ge = jnp.sum((x >= mid).astype(jnp.float32)) >= kf
        return (jnp.where(ge, mid, lo), jnp.where(ge, hi, mid))

    lo, hi = lax.fori_loop(0, 64, body, (lo0, hi0))

    gt = x > lo
    eq = x == lo
    gt_f = gt.astype(jnp.float32)
    eq_f = eq.astype(jnp.float32)
    need = kf - jnp.sum(gt_f)
    eq_excl = _excl_prefix(eq_f)
    sel = gt | (eq & (eq_excl < need))
    sel_f = sel.astype(jnp.float32)
    pos = _excl_prefix(sel_f).astype(jnp.int32)
    kk = lax.broadcasted_iota(jnp.int32, (SELK, x.shape[1]), 0)
    p_ref[0] = jnp.where((kk == pos) & sel, 1.0, 0.0)


def _select_onehot(scores):
    B, _, S = scores.shape
    return pl.pallas_call(
        _select_body,
        grid=(B,),
        compiler_params=_cp(1),
        in_specs=[pl.BlockSpec((1, 1, S), lambda b: (b, 0, 0))],
        out_specs=pl.BlockSpec((1, SELK, S), lambda b: (b, 0, 0)),
        out_shape=jax.ShapeDtypeStruct((B, SELK, S), jnp.float32),
    )(scores)


def _gather_body(p_ref, q_ref, k_ref, v_ref, qo, ko, vo):
    p = p_ref[0]
    qo[0] = jnp.dot(p, q_ref[0], preferred_element_type=jnp.float32)
    ko[0] = jnp.dot(p, k_ref[0], preferred_element_type=jnp.float32)
    vo[0] = jnp.dot(p, v_ref[0], preferred_element_type=jnp.float32)


def _gather(P, q, k, v):
    B, S, _ = q.shape
    CT = 256
    grid = (B, H // CT)
    qs, ks, vs = pl.pallas_call(
        _gather_body,
        grid=grid,
        compiler_params=_cp(2),
        in_specs=[pl.BlockSpec((1, SELK, S), lambda b, c: (b, 0, 0)),
                  pl.BlockSpec((1, S, CT), lambda b, c: (b, 0, c)),
                  pl.BlockSpec((1, S, CT), lambda b, c: (b, 0, c)),
                  pl.BlockSpec((1, S, CT), lambda b, c: (b, 0, c))],
        out_specs=[pl.BlockSpec((1, SELK, CT), lambda b, c: (b, 0, c))] * 3,
        out_shape=[jax.ShapeDtypeStruct((B, SELK, H), jnp.float32)] * 3,
    )(P, q, k, v)
    return qs, ks, vs


def _win_body(qlo, qhi, klo, khi, vlo, vhi, o_ref):
    q = jnp.concatenate([qlo[0], qhi[0]], axis=0)
    k = jnp.concatenate([klo[0], khi[0]], axis=0)
    v = jnp.concatenate([vlo[0], vhi[0]], axis=0)
    s = jnp.dot(q, k.T, preferred_element_type=jnp.float32) * SCALE
    o_ref[0] = jnp.dot(_softmax(s), v, preferred_element_type=jnp.float32)


def _window(q, k, v):
    B, S, _ = q.shape
    HW = WIN // 2
    lo = pl.BlockSpec((1, HW, H), lambda b, j: (b, j, 0))
    hi = pl.BlockSpec((1, HW, H), lambda b, j: (b, j + 1, 0))
    return pl.pallas_call(
        _win_body,
        grid=(B, S // WIN),
        compiler_params=_cp(2),
        in_specs=[lo, hi, lo, hi, lo, hi],
        out_specs=pl.BlockSpec((1, WIN, H), lambda b, j: (b, j, 0)),
        out_shape=jax.ShapeDtypeStruct((B, S, H), jnp.float32),
    )(q, q, k, k, v, v)


def _combine3_body(hs_ref, comp_ref, sel_ref, win_ref, wg, bg,
                   wo1, wo2, wo3, bo, o_ref):
    x = hs_ref[0]
    g = jax.nn.sigmoid(jnp.dot(x, wg[...], preferred_element_type=jnp.float32)
                       + bg[0])
    g = g / (jnp.sum(g, axis=-1, keepdims=True) + 1e-6)
    out = jnp.dot((comp_ref[0] * g[:, 0:1]).astype(jnp.bfloat16), wo1[...],
                  preferred_element_type=jnp.float32)
    out += jnp.dot((sel_ref[0] * g[:, 1:2]).astype(jnp.bfloat16), wo2[...],
                   preferred_element_type=jnp.float32)
    out += jnp.dot((win_ref[0] * g[:, 2:3]).astype(jnp.bfloat16), wo3[...],
                   preferred_element_type=jnp.float32)
    out += bo[0]
    r = out * 0.5 + x * 0.5
    mu = jnp.mean(r, axis=-1, keepdims=True)
    var = jnp.mean((r - mu) ** 2, axis=-1, keepdims=True)
    o_ref[0] = (r - mu) / jnp.sqrt(var + 1e-6)


def _combine1_body(hs_ref, win_ref, wg, bg, wo3, bo, o_ref):
    x = hs_ref[0]
    g = jax.nn.sigmoid(jnp.dot(x, wg[...], preferred_element_type=jnp.float32)
                       + bg[0])
    g = g / (jnp.sum(g, axis=-1, keepdims=True) + 1e-6)
    out = jnp.dot((win_ref[0] * g[:, 2:3]).astype(jnp.bfloat16), wo3[...],
                  preferred_element_type=jnp.float32) + bo[0]
    r = out * 0.5 + x * 0.5
    mu = jnp.mean(r, axis=-1, keepdims=True)
    var = jnp.mean((r - mu) ** 2, axis=-1, keepdims=True)
    o_ref[0] = (r - mu) / jnp.sqrt(var + 1e-6)


def _combine(hs, comp_out, sel_out, win_out, Wg, bg, Wo, bo):
    B, S, _ = hs.shape
    Wo16 = Wo.astype(jnp.bfloat16)
    Wo1, Wo2, Wo3 = Wo16[:H], Wo16[H:2 * H], Wo16[2 * H:]
    n_lo = SELK // TILE
    lo_spec = _row_spec(TILE)
    out_lo = pl.pallas_call(
        _combine3_body,
        grid=(B, n_lo),
        compiler_params=_cp(2),
        in_specs=[lo_spec, lo_spec, lo_spec, lo_spec,
                  _w_spec((H, 3)), _w_spec((1, 3)),
                  _w_spec((H, H)), _w_spec((H, H)), _w_spec((H, H)),
                  _w_spec((1, H))],
        out_specs=lo_spec,
        out_shape=jax.ShapeDtypeStruct((B, SELK, H), jnp.float32),
    )(hs[:, :SELK], comp_out, sel_out, win_out[:, :SELK], Wg, bg,
      Wo1, Wo2, Wo3, bo)
    n_hi = (S - SELK) // TILE
    out_hi = pl.pallas_call(
        _combine1_body,
        grid=(B, n_hi),
        compiler_params=_cp(2),
        in_specs=[lo_spec, lo_spec,
                  _w_spec((H, 3)), _w_spec((1, 3)),
                  _w_spec((H, H)), _w_spec((1, H))],
        out_specs=lo_spec,
        out_shape=jax.ShapeDtypeStruct((B, S - SELK, H), jnp.float32),
    )(hs[:, SELK:], win_out[:, SELK:], Wg, bg, Wo3, bo)
    return jnp.concatenate([out_lo, out_hi], axis=1)


def kernel(hidden_states, Wq, bq, Wk, bk, Wv, bv, Wo, bo, Wg, bg, Wc, bc, Ws, bs):
    B, S, _ = hidden_states.shape
    bq2, bk2, bv2 = bq[None, :], bk[None, :], bv[None, :]
    bs2 = bs[None, :]
    bg2 = bg[None, :]
    bo2 = bo[None, :]
    bc2 = bc[None, :]
    Wst = Ws.T
    Wq16 = Wq.astype(jnp.bfloat16)
    Wk16 = Wk.astype(jnp.bfloat16)
    Wv16 = Wv.astype(jnp.bfloat16)
    Wc16 = Wc.astype(jnp.bfloat16)

    q, k, v, scores = _qkv_score(hidden_states, Wq16, bq2, Wk16, bk2, Wv16, bv2,
                                 Wst, bs2)

    blocks = hidden_states.reshape(B, S // RATIO, RATIO * H)
    compressed = _compress(blocks, Wc16, bc2)
    qc, kc, vc = _qkv(compressed, Wq16, bq2, Wk16, bk2, Wv16, bv2)
    comp_out = _attn(qc, kc, vc)

    P = _select_onehot(scores)
    qs, ks, vs = _gather(P, q, k, v)
    sel_out = _attn(qs, ks, vs)

    win_out = _window(q, k, v)

    return _combine(hidden_states, comp_out, sel_out, win_out, Wg, bg2, Wo, bo2)


# speedup vs baseline: 1.2385x; 1.2385x over previous
import functools
import math

import jax
import jax.numpy as jnp
from jax import lax
from jax.experimental import pallas as pl
from jax.experimental.pallas import tpu as pltpu

H = 1024
RATIO = 4
SELK = 512
WIN = 256
SCALE = 1.0 / math.sqrt(H // 16)
TILE = 256
BF = jnp.bfloat16
F32 = jnp.float32


def _cp(ndims):
    return pltpu.CompilerParams(dimension_semantics=("parallel",) * ndims)


def _softmax(s):
    m = jnp.max(s, axis=-1, keepdims=True)
    e = jnp.exp(s - m)
    return e / jnp.sum(e, axis=-1, keepdims=True)


def _w_spec(shape):
    return pl.BlockSpec(shape, lambda b, t: (0,) * len(shape))


def _row_spec(n):
    return pl.BlockSpec((1, n, H), lambda b, t: (b, t, 0))


def _qkv_score_body(x_ref, wq, bq, wk, bk, wv, bv, ws, bs,
                    q_out, k_out, v_out, s_out):
    x = x_ref[0]
    x16 = x.astype(BF)
    q_out[0] = (jnp.dot(x16, wq[...], preferred_element_type=F32)
                + bq[0]).astype(BF)
    k_out[0] = (jnp.dot(x16, wk[...], preferred_element_type=F32)
                + bk[0]).astype(BF)
    v_out[0] = (jnp.dot(x16, wv[...], preferred_element_type=F32)
                + bv[0]).astype(BF)
    s_out[0] = lax.dot_general(ws[...], x, (((1,), (1,)), ((), ())),
                               preferred_element_type=F32) + bs[...]


def _qkv_score(x, Wq, bq, Wk, bk, Wv, bv, Wst, bs):
    B, S, _ = x.shape
    return pl.pallas_call(
        _qkv_score_body,
        grid=(B, S // TILE),
        compiler_params=_cp(2),
        in_specs=[
            _row_spec(TILE),
            _w_spec((H, H)), _w_spec((1, H)),
            _w_spec((H, H)), _w_spec((1, H)),
            _w_spec((H, H)), _w_spec((1, H)),
            _w_spec((1, H)), _w_spec((1, 1)),
        ],
        out_specs=[_row_spec(TILE), _row_spec(TILE), _row_spec(TILE),
                   pl.BlockSpec((1, 1, TILE), lambda b, t: (b, 0, t))],
        out_shape=[jax.ShapeDtypeStruct((B, S, H), BF)] * 3 + [
            jax.ShapeDtypeStruct((B, 1, S), F32)],
    )(x, Wq, bq, Wk, bk, Wv, bv, Wst, bs)


def _cproj_body(x_ref, wc, bc, out_ref):
    out_ref[0] = (jnp.dot(x_ref[0].astype(BF), wc[...],
                          preferred_element_type=F32) + bc[0]).astype(BF)


def _compress(blocks, Wc, bc):
    B, NB, D = blocks.shape
    return pl.pallas_call(
        _cproj_body,
        grid=(B, NB // TILE),
        compiler_params=_cp(2),
        in_specs=[pl.BlockSpec((1, TILE, D), lambda b, t: (b, t, 0)),
                  _w_spec((D, H)), _w_spec((1, H))],
        out_specs=_row_spec(TILE),
        out_shape=jax.ShapeDtypeStruct((B, NB, H), BF),
    )(blocks, Wc, bc)


def _comp_attn_body(c_ref, wq, bq, wk, bk, wv, bv, o_ref):
    c = c_ref[0]
    qc = jnp.dot(c, wq[...], preferred_element_type=F32) + bq[0]
    kc = jnp.dot(c, wk[...], preferred_element_type=F32) + bk[0]
    vc = jnp.dot(c, wv[...], preferred_element_type=F32) + bv[0]
    s = jnp.dot(qc, kc.T, preferred_element_type=F32) * SCALE
    o_ref[0] = jnp.dot(_softmax(s), vc, preferred_element_type=F32)


def _comp_attn(compressed, Wq, bq, Wk, bk, Wv, bv):
    B, N, _ = compressed.shape
    spec = pl.BlockSpec((1, N, H), lambda b: (b, 0, 0))
    w = lambda shape: pl.BlockSpec(shape, lambda b: (0,) * len(shape))
    return pl.pallas_call(
        _comp_attn_body,
        grid=(B,),
        compiler_params=_cp(1),
        in_specs=[spec,
                  w((H, H)), w((1, H)),
                  w((H, H)), w((1, H)),
                  w((H, H)), w((1, H))],
        out_specs=spec,
        out_shape=jax.ShapeDtypeStruct((B, N, H), F32),
    )(compressed, Wq, bq, Wk, bk, Wv, bv)


def _excl_prefix(f):
    S = f.shape[1]
    x = f
    k = 1
    while k < S:
        x = x + jnp.concatenate([jnp.zeros((1, k), f.dtype), x[:, :-k]], axis=1)
        k *= 2
    return x - f


def _select_body(s_ref, p_ref):
    x = s_ref[0]
    kf = float(SELK)

    lo0 = jnp.min(x)
    hi0 = jnp.max(x) + 1.0

    def body(_, lohi):
        lo, hi = lohi
        mid = (lo + hi) * 0.5
        ge = jnp.sum((x >= mid).astype(F32)) >= kf
        return (jnp.where(ge, mid, lo), jnp.where(ge, hi, mid))

    lo, hi = lax.fori_loop(0, 64, body, (lo0, hi0))

    gt = x > lo
    eq = x == lo
    gt_f = gt.astype(F32)
    eq_f = eq.astype(F32)
    need = kf - jnp.sum(gt_f)
    eq_excl = _excl_prefix(eq_f)
    sel = gt | (eq & (eq_excl < need))
    sel_f = sel.astype(F32)
    pos = _excl_prefix(sel_f).astype(jnp.int32)
    kk = lax.broadcasted_iota(jnp.int32, (SELK, x.shape[1]), 0)
    p_ref[0] = jnp.where((kk == pos) & sel, 1.0, 0.0).astype(BF)


def _select_onehot(scores):
    B, _, S = scores.shape
    return pl.pallas_call(
        _select_body,
        grid=(B,),
        compiler_params=_cp(1),
        in_specs=[pl.BlockSpec((1, 1, S), lambda b: (b, 0, 0))],
        out_specs=pl.BlockSpec((1, SELK, S), lambda b: (b, 0, 0)),
        out_shape=jax.ShapeDtypeStruct((B, SELK, S), BF),
    )(scores)


def _sel_attn_body(p_ref, q_ref, k_ref, v_ref, o_ref):
    p = p_ref[0]
    qs = jnp.dot(p, q_ref[0], preferred_element_type=F32)
    ks = jnp.dot(p, k_ref[0], preferred_element_type=F32)
    vs = jnp.dot(p, v_ref[0], preferred_element_type=F32)
    s = jnp.dot(qs, ks.T, preferred_element_type=F32) * SCALE
    o_ref[0] = jnp.dot(_softmax(s), vs, preferred_element_type=F32)


def _sel_attn(P, q, k, v):
    B, S, _ = q.shape
    full = pl.BlockSpec((1, S, H), lambda b: (b, 0, 0))
    return pl.pallas_call(
        _sel_attn_body,
        grid=(B,),
        compiler_params=_cp(1),
        in_specs=[pl.BlockSpec((1, SELK, S), lambda b: (b, 0, 0)),
                  full, full, full],
        out_specs=pl.BlockSpec((1, SELK, H), lambda b: (b, 0, 0)),
        out_shape=jax.ShapeDtypeStruct((B, SELK, H), F32),
    )(P, q, k, v)


def _win_attn(qlo, qhi, klo, khi, vlo, vhi):
    q = jnp.concatenate([qlo[0], qhi[0]], axis=0)
    k = jnp.concatenate([klo[0], khi[0]], axis=0)
    v = jnp.concatenate([vlo[0], vhi[0]], axis=0)
    s = jnp.dot(q, k.T, preferred_element_type=F32) * SCALE
    return jnp.dot(_softmax(s).astype(BF), v, preferred_element_type=F32)


def _finish(out, x):
    r = out * 0.5 + x * 0.5
    mu = jnp.mean(r, axis=-1, keepdims=True)
    var = jnp.mean((r - mu) ** 2, axis=-1, keepdims=True)
    return (r - mu) / jnp.sqrt(var + 1e-6)


def _gates(x, wg, bg):
    g = jax.nn.sigmoid(jnp.dot(x, wg[...], preferred_element_type=F32) + bg[0])
    return g / (jnp.sum(g, axis=-1, keepdims=True) + 1e-6)


def _combine3_body(qlo, qhi, klo, khi, vlo, vhi, hs_ref, comp_ref, sel_ref,
                   wg, bg, wo1, wo2, wo3, bo, o_ref):
    x = hs_ref[0]
    g = _gates(x, wg, bg)
    win = _win_attn(qlo, qhi, klo, khi, vlo, vhi)
    out = jnp.dot((win * g[:, 2:3]).astype(BF), wo3[...],
                  preferred_element_type=F32)
    out += jnp.dot((comp_ref[0] * g[:, 0:1]).astype(BF), wo1[...],
                   preferred_element_type=F32)
    out += jnp.dot((sel_ref[0] * g[:, 1:2]).astype(BF), wo2[...],
                   preferred_element_type=F32)
    out += bo[0]
    o_ref[0] = _finish(out, x)


def _combine1_body(qlo, qhi, klo, khi, vlo, vhi, hs_ref,
                   wg, bg, wo3, bo, o_ref):
    x = hs_ref[0]
    g = _gates(x, wg, bg)
    win = _win_attn(qlo, qhi, klo, khi, vlo, vhi)
    out = jnp.dot((win * g[:, 2:3]).astype(BF), wo3[...],
                  preferred_element_type=F32) + bo[0]
    o_ref[0] = _finish(out, x)


def _combine(hs, q, k, v, comp_out, sel_out, Wg, bg, Wo1, Wo2, Wo3, bo):
    B, S, _ = hs.shape
    HW = WIN // 2
    lo = pl.BlockSpec((1, HW, H), lambda b, j: (b, j, 0))
    hi = pl.BlockSpec((1, HW, H), lambda b, j: (b, j + 1, 0))
    lo2 = pl.BlockSpec((1, HW, H), lambda b, j: (b, j + 2, 0))
    hi2 = pl.BlockSpec((1, HW, H), lambda b, j: (b, j + 3, 0))
    tile = _row_spec(TILE)
    out_lo = pl.pallas_call(
        _combine3_body,
        grid=(B, SELK // WIN),
        compiler_params=_cp(2),
        in_specs=[lo, hi, lo, hi, lo, hi, tile, tile, tile,
                  _w_spec((H, 3)), _w_spec((1, 3)),
                  _w_spec((H, H)), _w_spec((H, H)), _w_spec((H, H)),
                  _w_spec((1, H))],
        out_specs=tile,
        out_shape=jax.ShapeDtypeStruct((B, SELK, H), F32),
    )(q, q, k, k, v, v, hs[:, :SELK], comp_out, sel_out, Wg, bg,
      Wo1, Wo2, Wo3, bo)
    out_hi = pl.pallas_call(
        _combine1_body,
        grid=(B, (S - SELK) // WIN),
        compiler_params=_cp(2),
        in_specs=[lo2, hi2, lo2, hi2, lo2, hi2, tile,
                  _w_spec((H, 3)), _w_spec((1, 3)),
                  _w_spec((H, H)), _w_spec((1, H))],
        out_specs=tile,
        out_shape=jax.ShapeDtypeStruct((B, S - SELK, H), F32),
    )(q, q, k, k, v, v, hs[:, SELK:], Wg, bg, Wo3, bo)
    return jnp.concatenate([out_lo, out_hi], axis=1)


def kernel(hidden_states, Wq, bq, Wk, bk, Wv, bv, Wo, bo, Wg, bg, Wc, bc, Ws, bs):
    B, S, _ = hidden_states.shape
    bq2, bk2, bv2 = bq[None, :], bk[None, :], bv[None, :]
    bs2, bg2, bo2, bc2 = bs[None, :], bg[None, :], bo[None, :], bc[None, :]
    Wst = Ws.T
    Wq16, Wk16, Wv16 = Wq.astype(BF), Wk.astype(BF), Wv.astype(BF)
    Wc16 = Wc.astype(BF)
    Wo16 = Wo.astype(BF)
    Wo1, Wo2, Wo3 = Wo16[:H], Wo16[H:2 * H], Wo16[2 * H:]

    q, k, v, scores = _qkv_score(hidden_states, Wq16, bq2, Wk16, bk2,
                                 Wv16, bv2, Wst, bs2)

    blocks = hidden_states.reshape(B, S // RATIO, RATIO * H)
    compressed = _compress(blocks, Wc16, bc2)
    comp_out = _comp_attn(compressed, Wq16, bq2, Wk16, bk2, Wv16, bv2)

    P = _select_onehot(scores)
    sel_out = _sel_attn(P, q, k, v)

    return _combine(hidden_states, q, k, v, comp_out, sel_out,
                    Wg, bg2, Wo1, Wo2, Wo3, bo2)


# speedup vs baseline: 1.4467x; 1.1681x over previous
import functools
import math

import jax
import jax.numpy as jnp
from jax import lax
from jax.experimental import pallas as pl
from jax.experimental.pallas import tpu as pltpu

H = 1024
RATIO = 4
SELK = 512
WIN = 256
SCALE = 1.0 / math.sqrt(H // 16)
TILE = 256
BF = jnp.bfloat16
F32 = jnp.float32


def _cp(ndims):
    return pltpu.CompilerParams(dimension_semantics=("parallel",) * ndims)


def _softmax(s):
    m = jnp.max(s, axis=-1, keepdims=True)
    e = jnp.exp(s - m)
    return e / jnp.sum(e, axis=-1, keepdims=True)


def _w_spec(shape):
    return pl.BlockSpec(shape, lambda b, t: (0,) * len(shape))


def _row_spec(n):
    return pl.BlockSpec((1, n, H), lambda b, t: (b, t, 0))


def _qkv_score_body(x_ref, wq, bq, wk, bk, wv, bv, ws, bs,
                    q_out, k_out, v_out, s_out):
    x = x_ref[0]
    x16 = x.astype(BF)
    q_out[0] = (jnp.dot(x16, wq[...], preferred_element_type=F32)
                + bq[0]).astype(BF)
    k_out[0] = (jnp.dot(x16, wk[...], preferred_element_type=F32)
                + bk[0]).astype(BF)
    v_out[0] = (jnp.dot(x16, wv[...], preferred_element_type=F32)
                + bv[0]).astype(BF)
    s_out[0] = lax.dot_general(ws[...], x, (((1,), (1,)), ((), ())),
                               preferred_element_type=F32) + bs[...]


def _qkv_score(x, Wq, bq, Wk, bk, Wv, bv, Wst, bs):
    B, S, _ = x.shape
    return pl.pallas_call(
        _qkv_score_body,
        grid=(B, S // TILE),
        compiler_params=_cp(2),
        in_specs=[
            _row_spec(TILE),
            _w_spec((H, H)), _w_spec((1, H)),
            _w_spec((H, H)), _w_spec((1, H)),
            _w_spec((H, H)), _w_spec((1, H)),
            _w_spec((1, H)), _w_spec((1, 1)),
        ],
        out_specs=[_row_spec(TILE), _row_spec(TILE), _row_spec(TILE),
                   pl.BlockSpec((1, 1, TILE), lambda b, t: (b, 0, t))],
        out_shape=[jax.ShapeDtypeStruct((B, S, H), BF)] * 3 + [
            jax.ShapeDtypeStruct((B, 1, S), F32)],
    )(x, Wq, bq, Wk, bk, Wv, bv, Wst, bs)


def _comp_branch_body(blk_ref, wc, bc, wq, bq, wk, bk, wv, bv, o_ref):
    c = (jnp.dot(blk_ref[0].astype(BF), wc[...],
                 preferred_element_type=F32) + bc[0]).astype(BF)
    qc = jnp.dot(c, wq[...], preferred_element_type=F32) + bq[0]
    kc = jnp.dot(c, wk[...], preferred_element_type=F32) + bk[0]
    vc = jnp.dot(c, wv[...], preferred_element_type=F32) + bv[0]
    s = jnp.dot(qc, kc.T, preferred_element_type=F32) * SCALE
    o_ref[0] = jnp.dot(_softmax(s).astype(BF), vc.astype(BF),
                       preferred_element_type=F32).astype(BF)


def _comp_branch(blocks, Wc, bc, Wq, bq, Wk, bk, Wv, bv):
    B, NB, D = blocks.shape
    w = lambda shape: pl.BlockSpec(shape, lambda b: (0,) * len(shape))
    return pl.pallas_call(
        _comp_branch_body,
        grid=(B,),
        compiler_params=_cp(1),
        in_specs=[pl.BlockSpec((1, NB, D), lambda b: (b, 0, 0)),
                  w((D, H)), w((1, H)),
                  w((H, H)), w((1, H)),
                  w((H, H)), w((1, H)),
                  w((H, H)), w((1, H))],
        out_specs=pl.BlockSpec((1, NB, H), lambda b: (b, 0, 0)),
        out_shape=jax.ShapeDtypeStruct((B, NB, H), BF),
    )(blocks, Wc, bc, Wq, bq, Wk, bk, Wv, bv)


def _excl_prefix(f):
    S = f.shape[1]
    x = f
    k = 1
    while k < S:
        x = x + jnp.concatenate([jnp.zeros((1, k), f.dtype), x[:, :-k]], axis=1)
        k *= 2
    return x - f


def _sel_branch_body(s_ref, q_ref, k_ref, v_ref, o_ref):
    x = s_ref[0]
    kf = float(SELK)

    lo0 = jnp.min(x)
    hi0 = jnp.max(x) + 1.0

    def body(_, lohi):
        lo, hi = lohi
        mid = (lo + hi) * 0.5
        ge = jnp.sum((x >= mid).astype(F32)) >= kf
        return (jnp.where(ge, mid, lo), jnp.where(ge, hi, mid))

    lo, hi = lax.fori_loop(0, 64, body, (lo0, hi0))

    gt = x > lo
    eq = x == lo
    gt_f = gt.astype(F32)
    eq_f = eq.astype(F32)
    need = kf - jnp.sum(gt_f)
    eq_excl = _excl_prefix(eq_f)
    sel = gt | (eq & (eq_excl < need))
    sel_f = sel.astype(F32)
    pos = _excl_prefix(sel_f).astype(jnp.int32)
    kk = lax.broadcasted_iota(jnp.int32, (SELK, x.shape[1]), 0)
    p = jnp.where((kk == pos) & sel, 1.0, 0.0).astype(BF)

    qs = jnp.dot(p, q_ref[0], preferred_element_type=F32)
    ks = jnp.dot(p, k_ref[0], preferred_element_type=F32)
    vs = jnp.dot(p, v_ref[0], preferred_element_type=F32)
    s = jnp.dot(qs.astype(BF), ks.astype(BF).T, preferred_element_type=F32) * SCALE
    o_ref[0] = jnp.dot(_softmax(s).astype(BF), vs.astype(BF),
                       preferred_element_type=F32).astype(BF)


def _sel_branch(scores, q, k, v):
    B, S, _ = q.shape
    full = pl.BlockSpec((1, S, H), lambda b: (b, 0, 0))
    return pl.pallas_call(
        _sel_branch_body,
        grid=(B,),
        compiler_params=_cp(1),
        in_specs=[pl.BlockSpec((1, 1, S), lambda b: (b, 0, 0)),
                  full, full, full],
        out_specs=pl.BlockSpec((1, SELK, H), lambda b: (b, 0, 0)),
        out_shape=jax.ShapeDtypeStruct((B, SELK, H), BF),
    )(scores, q, k, v)


def _win_attn(qlo, qhi, klo, khi, vlo, vhi):
    q = jnp.concatenate([qlo[0], qhi[0]], axis=0)
    k = jnp.concatenate([klo[0], khi[0]], axis=0)
    v = jnp.concatenate([vlo[0], vhi[0]], axis=0)
    s = jnp.dot(q, k.T, preferred_element_type=F32) * SCALE
    return jnp.dot(_softmax(s).astype(BF), v, preferred_element_type=F32)


def _finish(out, x):
    r = out * 0.5 + x * 0.5
    mu = jnp.mean(r, axis=-1, keepdims=True)
    var = jnp.mean((r - mu) ** 2, axis=-1, keepdims=True)
    return (r - mu) / jnp.sqrt(var + 1e-6)


def _gates(x, wg, bg):
    g = jax.nn.sigmoid(jnp.dot(x, wg[...], preferred_element_type=F32) + bg[0])
    return g / (jnp.sum(g, axis=-1, keepdims=True) + 1e-6)


def _combine_body(qlo, qhi, klo, khi, vlo, vhi, hs_ref, comp_ref, sel_ref,
                  wg, bg, wo1, wo2, wo3, bo, o_ref, acc_ref):
    j = pl.program_id(1)
    x = hs_ref[0]
    g = _gates(x, wg, bg)
    win = _win_attn(qlo, qhi, klo, khi, vlo, vhi)
    acc_ref[...] = jnp.dot((win * g[:, 2:3]).astype(BF), wo3[...],
                           preferred_element_type=F32) + bo[0]

    @pl.when(j < SELK // WIN)
    def _():
        extra = jnp.dot((comp_ref[0].astype(F32) * g[:, 0:1]).astype(BF),
                        wo1[...], preferred_element_type=F32)
        extra += jnp.dot((sel_ref[0].astype(F32) * g[:, 1:2]).astype(BF),
                         wo2[...], preferred_element_type=F32)
        acc_ref[...] += extra

    o_ref[0] = _finish(acc_ref[...], x)


def _combine(hs, q, k, v, comp_out, sel_out, Wg, bg, Wo1, Wo2, Wo3, bo):
    B, S, _ = hs.shape
    HW = WIN // 2
    NJ = S // WIN
    lo = pl.BlockSpec((1, HW, H), lambda b, j: (b, j, 0))
    hi = pl.BlockSpec((1, HW, H), lambda b, j: (b, j + 1, 0))
    tile = _row_spec(WIN)
    cs_tile = pl.BlockSpec((1, WIN, H), lambda b, j: (b, jnp.minimum(j, SELK // WIN - 1), 0))
    return pl.pallas_call(
        _combine_body,
        grid=(B, NJ),
        compiler_params=_cp(2),
        in_specs=[lo, hi, lo, hi, lo, hi, tile, cs_tile, cs_tile,
                  _w_spec((H, 3)), _w_spec((1, 3)),
                  _w_spec((H, H)), _w_spec((H, H)), _w_spec((H, H)),
                  _w_spec((1, H))],
        out_specs=tile,
        out_shape=jax.ShapeDtypeStruct((B, S, H), F32),
        scratch_shapes=[pltpu.VMEM((WIN, H), F32)],
    )(q, q, k, k, v, v, hs, comp_out, sel_out, Wg, bg, Wo1, Wo2, Wo3, bo)


def kernel(hidden_states, Wq, bq, Wk, bk, Wv, bv, Wo, bo, Wg, bg, Wc, bc, Ws, bs):
    B, S, _ = hidden_states.shape
    bq2, bk2, bv2 = bq[None, :], bk[None, :], bv[None, :]
    bs2, bg2, bo2, bc2 = bs[None, :], bg[None, :], bo[None, :], bc[None, :]
    Wst = Ws.T
    Wq16, Wk16, Wv16 = Wq.astype(BF), Wk.astype(BF), Wv.astype(BF)
    Wc16 = Wc.astype(BF)
    Wo16 = Wo.astype(BF)
    Wo1, Wo2, Wo3 = Wo16[:H], Wo16[H:2 * H], Wo16[2 * H:]

    q, k, v, scores = _qkv_score(hidden_states, Wq16, bq2, Wk16, bk2,
                                 Wv16, bv2, Wst, bs2)

    blocks = hidden_states.reshape(B, S // RATIO, RATIO * H)
    comp_out = _comp_branch(blocks, Wc16, bc2, Wq16, bq2, Wk16, bk2, Wv16, bv2)

    sel_out = _sel_branch(scores, q, k, v)

    return _combine(hidden_states, q, k, v, comp_out, sel_out,
                    Wg, bg2, Wo1, Wo2, Wo3, bo2)
